# Initial kernel scaffold; baseline (speedup 1.0000x reference)
import jax
import jax.numpy as jnp
from jax.experimental import pallas as pl


def kernel(normed_stream, params):
    raise NotImplementedError("write your pallas kernel here")


# speedup vs baseline: 12.5143x; 12.5143x over previous
import math
import jax
import jax.numpy as jnp
from jax.experimental import pallas as pl
from jax.experimental.pallas import tpu as pltpu

_D = 1024; _DK = 16; _DV = 16; _DKP = 2 * _DK
_EQ = 8; _EK = 8; _NM = 4; _R = max(_DK // 4, 1)
_DALPHA = int(_DK * 1.618); _CONCAT = _EK * _DV; _DPG = int(_CONCAT * 0.618)
_Q_TP = 0.8; _Q_MK = 2; _KV_TP = 0.8; _KV_MK = 2


def _sinkhorn_knopp(M, n_iters=6):
    M = jnp.exp(M)
    for _ in range(n_iters):
        M = M / M.sum(axis=-1, keepdims=True)
        M = M / M.sum(axis=-2, keepdims=True)
    return M


def _rms_norm(x, scale):
    return x * jax.lax.rsqrt(jnp.mean(x * x, axis=-1, keepdims=True) + 1e-6) * scale


def _top_prob_max_k(logits, threshold, max_k):
    probs = jax.nn.softmax(logits, axis=-1)
    sorted_idx = jnp.argsort(-probs, axis=-1)
    sorted_p = jnp.take_along_axis(probs, sorted_idx, axis=-1)
    cumsum = jnp.cumsum(sorted_p, axis=-1)
    ar = jnp.arange(logits.shape[-1])
    mask = (cumsum - sorted_p < threshold) & (ar < max_k)
    mask = mask.at[..., 0].set(True)
    selected = sorted_p * mask.astype(sorted_p.dtype)
    inv = jnp.argsort(sorted_idx, axis=-1)
    return jnp.take_along_axis(selected, inv, axis=-1)


def _mhc_forward(x_normed_all, mp, e):
    x = x_normed_all * mp['scale'][e]
    B, T, _ = x.shape
    H_pre = jax.nn.sigmoid(mp['a_pre'][e] * (x @ mp['phi_pre'][e]) + mp['b_pre'][e])
    H_post = 2.0 * jax.nn.sigmoid(mp['a_post'][e] * (x @ mp['phi_post'][e]) + mp['b_post'][e])
    H_res_raw = (mp['a_res'][e] * (x @ mp['phi_res'][e])).reshape(B, T, _NM, _NM) + mp['b_res'][e]
    return _sinkhorn_knopp(H_res_raw), H_pre, H_post


def _apply_pope(x, positions, freqs, delta_raw, is_query):
    mu = jax.nn.softplus(x)
    phi = positions[:, None] * freqs[None, :]
    if not is_query:
        phi = phi - 2.0 * math.pi * jax.nn.sigmoid(delta_raw)
    return jnp.concatenate([mu * jnp.cos(phi), mu * jnp.sin(phi)], axis=-1)


def _kda_kernel(q_ref, k_ref, v_ref, a_ref, b_ref, out_ref, S_ref):
    @pl.when(pl.program_id(0) == 0)
    def _init():
        S_ref[...] = jnp.zeros_like(S_ref)

    CT = q_ref.shape[0]

    def step(t, carry):
        q_t = q_ref[pl.ds(t, 1)][0]
        k_t = k_ref[pl.ds(t, 1)][0]
        v_t = v_ref[pl.ds(t, 1)][0]
        a_t = a_ref[pl.ds(t, 1)][0]
        b_t = b_ref[pl.ds(t, 1)][0]
        S = S_ref[...]
        aS = a_t[:, :, None] * S
        kt_aS = jnp.sum(k_t[:, :, None] * aS, axis=1)
        bk = b_t[:, None, None] * k_t[:, :, None]
        S_new = aS + bk * (v_t[:, None, :] - kt_aS[:, None, :])
        out_t = jnp.sum(q_t[:, :, None] * S_new, axis=1)
        out_ref[pl.ds(t, 1)] = out_t[None]
        S_ref[...] = S_new
        return carry

    jax.lax.fori_loop(0, CT, step, 0)


def _kda_all_experts(q_total, k_all, v_all, a_all, b_all):
    EK, B, T, DKP = k_all.shape
    DV = v_all.shape[-1]
    EB = EK * B
    q_tiled = jnp.broadcast_to(q_total[None], (EK, B, T, DKP))

    def to_tfirst(x):
        return jnp.transpose(x, (2, 0, 1, 3)).reshape(T, EB, x.shape[-1])

    q_in = to_tfirst(q_tiled)
    k_in = to_tfirst(k_all)
    v_in = to_tfirst(v_all)
    a_in = to_tfirst(a_all)
    b_in = jnp.transpose(b_all, (2, 0, 1)).reshape(T, EB)

    CT = 256
    nchunks = T // CT
    out = pl.pallas_call(
        _kda_kernel,
        grid=(nchunks,),
        in_specs=[
            pl.BlockSpec((CT, EB, DKP), lambda i: (i, 0, 0)),
            pl.BlockSpec((CT, EB, DKP), lambda i: (i, 0, 0)),
            pl.BlockSpec((CT, EB, DV), lambda i: (i, 0, 0)),
            pl.BlockSpec((CT, EB, DKP), lambda i: (i, 0, 0)),
            pl.BlockSpec((CT, EB), lambda i: (i, 0)),
        ],
        out_specs=pl.BlockSpec((CT, EB, DV), lambda i: (i, 0, 0)),
        scratch_shapes=[pltpu.VMEM((EB, DKP, DV), jnp.float32)],
        out_shape=jax.ShapeDtypeStruct((T, EB, DV), q_total.dtype),
    )(q_in, k_in, v_in, a_in, b_in)
    return jnp.transpose(out.reshape(T, EK, B, DV), (1, 2, 0, 3))


def kernel(normed_stream, params):
    stream = normed_stream
    B, n, T, d = stream.shape
    positions = jnp.arange(T, dtype=stream.dtype)
    route_input = stream.mean(axis=1)
    gate_q = _top_prob_max_k(route_input @ params['router_q'], _Q_TP, _Q_MK)
    gate_kv = _top_prob_max_k(route_input @ params['router_kv'], _KV_TP, _KV_MK)

    x_flat = jnp.transpose(stream, (0, 2, 1, 3)).reshape(B, T, n * d)
    x_base = x_flat * jax.lax.rsqrt(jnp.mean(x_flat * x_flat, axis=-1, keepdims=True) + 1e-6)

    q_total = jnp.zeros((B, T, _DKP), dtype=stream.dtype)
    H_post_acc = jnp.zeros((B, T, n), dtype=stream.dtype)
    H_res_acc = jnp.zeros((B, T, n, n), dtype=stream.dtype)
    for e in range(_EQ):
        H_res, H_pre, H_post = _mhc_forward(x_base, params['mhc_q'], e)
        h = jnp.einsum('btn,bntd->btd', H_pre, stream)
        q_e = h @ params['W_q'] + (h @ params['lora_A_q'][e]) @ params['lora_B_q'][e]
        q_p = _apply_pope(q_e, positions, params['freqs'], params['pope_delta'], True)
        g = gate_q[..., e:e + 1]
        q_total = q_total + g * q_p
        H_post_acc = H_post_acc + g * H_post
        H_res_acc = H_res_acc + g[..., None] * H_res

    k_list, v_list, a_list, b_list = [], [], [], []
    for e in range(_EK):
        H_res, H_pre, H_post = _mhc_forward(x_base, params['mhc_kv'], e)
        h = jnp.einsum('btn,bntd->btd', H_pre, stream)
        k_e = h @ params['W_k'] + (h @ params['lora_A_k'][e]) @ params['lora_B_k'][e]
        k_p = _apply_pope(k_e, positions, params['freqs'], params['pope_delta'], False)
        v_e = h @ params['W_v'] + (h @ params['lora_A_v'][e]) @ params['lora_B_v'][e]
        alpha = jax.nn.sigmoid(jax.nn.silu(h @ params['alpha_up'][e]) @ params['alpha_down'][e])
        beta = jax.nn.sigmoid(jax.nn.silu(h @ params['beta_up'][e]) @ params['beta_down'][e])[..., 0]
        k_list.append(k_p); v_list.append(v_e); a_list.append(alpha); b_list.append(beta)
        g = gate_kv[..., e:e + 1]
        H_post_acc = H_post_acc + g * H_post
        H_res_acc = H_res_acc + g[..., None] * H_res

    o_all = _kda_all_experts(q_total,
                             jnp.stack(k_list), jnp.stack(v_list),
                             jnp.stack(a_list), jnp.stack(b_list))
    out_heads = [gate_kv[..., e:e + 1] * o_all[e] for e in range(_EK)]
    out_cat = jnp.concatenate(out_heads, axis=-1)
    pre = jax.nn.sigmoid(route_input @ params['W_pre'])
    y = (out_cat * pre) @ params['W_o']
    pg = jax.nn.sigmoid(jax.nn.silu(route_input @ params['W_pg1']) @ params['W_pg2'])
    y = y * pg
    denom = float(_EQ + _EK)
    H_post_avg = H_post_acc / denom
    H_res_avg = H_res_acc / denom
    mixed = jnp.einsum('btij,bjtd->bitd', H_res_avg, stream)
    return mixed + jnp.transpose(H_post_avg, (0, 2, 1))[..., None] * y[:, None, :, :]
